# Initial kernel scaffold; baseline (speedup 1.0000x reference)
import jax
import jax.numpy as jnp
from jax.experimental import pallas as pl


def kernel(input, hg_idx, related_items, label, uid, params):
    raise NotImplementedError("write your pallas kernel here")


# speedup vs baseline: 1.1593x; 1.1593x over previous
import functools

import jax
import jax.numpy as jnp
import numpy as np
from jax.experimental import pallas as pl
from jax.experimental.pallas import tpu as pltpu

BSZ = 64
LENS = 300
EMB = 128
HALF = EMB // 2
D = EMB + HALF
LAYERS = 2
N_NODES = BSZ * LENS
N_EDGES_HG = N_NODES
E_INC = 307200
N_ITEMS = 20000
TEXT_DIM = 384
IMG_DIM = 2048
HEADS = 4
DH = D // HEADS


def _make_pe():
    position = np.arange(LENS)[:, None].astype(np.float64)
    div_term = np.exp(np.arange(0, EMB, 2) * (-np.log(10000.0) / EMB))
    pe = np.zeros((LENS + 1, EMB))
    pe[1:, 0::2] = np.sin(position * div_term)
    pe[1:, 1::2] = np.cos(position * div_term)
    return jnp.asarray(pe, dtype=jnp.float32)


_ROWS_BLK = 400


def _proj_body(text_ref, img_ref, w1_ref, b1_ref, w2_ref, b2_ref, out_ref):
    t = jnp.dot(text_ref[...], w1_ref[...], preferred_element_type=jnp.float32)
    i = jnp.dot(img_ref[...], w2_ref[...], preferred_element_type=jnp.float32)
    out_ref[:, :EMB] = t + b1_ref[...]
    out_ref[:, EMB:] = i + b2_ref[...]


def _project_tables(text_table, img_table, W1, b1, W2, b2):
    grid = (N_ITEMS // _ROWS_BLK,)
    return pl.pallas_call(
        _proj_body,
        grid=grid,
        in_specs=[
            pl.BlockSpec((_ROWS_BLK, TEXT_DIM), lambda i: (i, 0)),
            pl.BlockSpec((_ROWS_BLK, IMG_DIM), lambda i: (i, 0)),
            pl.BlockSpec((TEXT_DIM, EMB), lambda i: (0, 0)),
            pl.BlockSpec((EMB,), lambda i: (0,)),
            pl.BlockSpec((IMG_DIM, EMB), lambda i: (0, 0)),
            pl.BlockSpec((EMB,), lambda i: (0,)),
        ],
        out_specs=pl.BlockSpec((_ROWS_BLK, 2 * EMB), lambda i: (i, 0)),
        out_shape=jax.ShapeDtypeStruct((N_ITEMS, 2 * EMB), jnp.float32),
    )(text_table, img_table, W1, b1, W2, b2)


def _hgconv2(x, node, edge, theta_t, bias_t, theta_i, bias_i, Binv, Dinv):
    xt = x[:, :EMB] @ theta_t
    xi = x[:, EMB:] @ theta_i
    y = jnp.concatenate([xt, xi], axis=1)
    e_feat = jax.ops.segment_sum(y[node], edge, num_segments=N_EDGES_HG) * Binv[:, None]
    out = jax.ops.segment_sum(e_feat[edge], node, num_segments=N_NODES) * Dinv[:, None]
    bias = jnp.concatenate([bias_t, bias_i])
    return out + bias[None, :]


def kernel(input, hg_idx, related_items, label, uid, params):
    p = params
    node = hg_idx[0]
    edge = hg_idx[1]

    P = _project_tables(p['text_table'], p['img_table'], p['W1'], p['b1'],
                        p['W2'], p['b2'])
    x = P[related_items]

    pe = _make_pe()
    pos = pe[jnp.arange(LENS)] * 0.002
    pos2 = jnp.tile(pos, (BSZ, 2))
    x = x + pos2

    ones = jnp.ones(node.shape[0], dtype=jnp.float32)
    Dn = jax.ops.segment_sum(ones, node, num_segments=N_NODES)
    Be = jax.ops.segment_sum(ones, edge, num_segments=N_EDGES_HG)
    Binv = jnp.where(Be > 0, 1.0 / Be, 0.0)
    Dinv = jnp.where(Dn > 0, 1.0 / Dn, 0.0)

    for l in range(LAYERS):
        x = _hgconv2(x, node, edge, p['theta_t%d' % l], p['bias_t%d' % l],
                     p['theta_i%d' % l], p['bias_i%d' % l], Binv, Dinv)

    sel = jnp.arange(BSZ) * LENS
    tg0 = x[sel, :EMB]
    ig0 = x[sel, EMB:]

    user = p['user_table'][uid]
    text_user = jnp.concatenate([tg0, user], axis=1)
    img_user = jnp.concatenate([ig0, user], axis=1)
    tiu = jnp.concatenate([text_user, img_user], axis=-1) @ p['W3'] + p['b3']

    q = tiu[:, None, :]
    kv = jnp.concatenate([text_user[:, None, :], img_user[:, None, :]], axis=1)
    Q = (q @ p['Wq'] + p['bq']).reshape(BSZ, 1, HEADS, DH).transpose(0, 2, 1, 3)
    K = (kv @ p['Wk'] + p['bk']).reshape(BSZ, 2, HEADS, DH).transpose(0, 2, 1, 3)
    V = (kv @ p['Wv'] + p['bv']).reshape(BSZ, 2, HEADS, DH).transpose(0, 2, 1, 3)
    att = jax.nn.softmax(Q @ K.transpose(0, 1, 3, 2) / jnp.sqrt(jnp.float32(DH)), axis=-1)
    ao = (att @ V).transpose(0, 2, 1, 3).reshape(BSZ, 1, D) @ p['Wo'] + p['bo']

    def layer_norm(x, g, b):
        m = jnp.mean(x, axis=-1, keepdims=True)
        v = jnp.var(x, axis=-1, keepdims=True)
        return (x - m) / jnp.sqrt(v + 1e-5) * g + b

    xh = layer_norm(q + ao, p['ln1_g'], p['ln1_b'])
    ff = jax.nn.relu(xh @ p['Wf1'] + p['bf1']) @ p['Wf2'] + p['bf2']
    xh = layer_norm(xh + ff, p['ln2_g'], p['ln2_b'])
    out = xh[:, 0, :]
    out = jax.nn.relu(out @ p['dW'] + p['db'])
    out = out @ p['W4'] + p['b4']
    return out


# speedup vs baseline: 4.8612x; 4.1934x over previous
import functools

import jax
import jax.numpy as jnp
import numpy as np
from jax import lax
from jax.experimental import pallas as pl
from jax.experimental.pallas import tpu as pltpu
from jax.experimental.pallas import tpu_sc as plsc

BSZ = 64
LENS = 300
EMB = 128
HALF = EMB // 2
D = EMB + HALF
LAYERS = 2
N_NODES = BSZ * LENS
E_INC = 307200
N_ITEMS = 20000
TEXT_DIM = 384
IMG_DIM = 2048
HEADS = 4
DH = D // HEADS

FW = 2 * EMB
NCH = 2
CW = FW // NCH

E2D_R, E2D_C = 3200, 96
RPT = E2D_R // 16
HROWS = N_NODES // 2
ACC_R = HROWS + 16
TPR = HROWS // 16
SEG = 40
NSEG = RPT // SEG
NBUF = 2

G2D_R, G2D_C = 256, 80
GRPW = G2D_R // 32


def _make_pos_rep():
    position = np.arange(LENS)[:, None].astype(np.float64)
    div_term = np.exp(np.arange(0, EMB, 2) * (-np.log(10000.0) / EMB))
    pe = np.zeros((LENS + 1, EMB))
    pe[1:, 0::2] = np.sin(position * div_term)
    pe[1:, 1::2] = np.cos(position * div_term)
    pos = pe[0:LENS].astype(np.float32) * 0.002
    return np.tile(pos, (4, 2))


_POS_REP_NP = _make_pos_rep()

_SC_MESH = plsc.VectorSubcoreMesh(core_axis_name="c", subcore_axis_name="s")


_PROJ_BLK = 400


def _proj_body(text_ref, img_ref, w1_ref, b1_ref, w2_ref, b2_ref, out_ref):
    t = jnp.dot(text_ref[...], w1_ref[...], preferred_element_type=jnp.float32, precision=lax.Precision.HIGHEST)
    i = jnp.dot(img_ref[...], w2_ref[...], preferred_element_type=jnp.float32, precision=lax.Precision.HIGHEST)
    out_ref[:, :EMB] = t + b1_ref[...]
    out_ref[:, EMB:] = i + b2_ref[...]


@functools.partial(
    pl.kernel,
    out_type=jax.ShapeDtypeStruct((G2D_R, G2D_C, FW), jnp.float32),
    mesh=_SC_MESH,
    scratch_types=[
        pltpu.VMEM((GRPW, G2D_C), jnp.int32),
        pltpu.VMEM((G2D_C, FW), jnp.float32),
        pltpu.VMEM((G2D_C, FW), jnp.float32),
        pltpu.SemaphoreType.DMA,
        pltpu.SemaphoreType.DMA,
    ],
)
def _sc_gather(p_hbm, ri_hbm, x_hbm, idx_v, buf0, buf1, sem0, sem1):
    cid = lax.axis_index("c")
    sid = lax.axis_index("s")
    wid = cid * 16 + sid
    pltpu.sync_copy(ri_hbm.at[pl.ds(wid * GRPW, GRPW)], idx_v)
    bufs = (buf0, buf1)
    sems = (sem0, sem1)
    for k in range(2):
        pltpu.async_copy(p_hbm.at[idx_v.at[k]], bufs[k], sems[k])

    def blk(jb, _):
        for k in range(2):
            j = jb * 2 + k
            pltpu.make_async_copy(p_hbm.at[idx_v.at[k]], bufs[k], sems[k]).wait()
            pltpu.sync_copy(bufs[k], x_hbm.at[wid * GRPW + j])
            nj = j + 2

            @pl.when(nj < GRPW)
            def _():
                pltpu.async_copy(p_hbm.at[idx_v.at[nj]], bufs[k], sems[k])
        return 0

    lax.fori_loop(0, GRPW // 2, blk, 0)


_SEG_SCRATCH = [
    pltpu.VMEM_SHARED((ACC_R, CW), jnp.float32),
    pltpu.VMEM((SEG, E2D_C), jnp.int32),
    pltpu.VMEM((SEG, E2D_C), jnp.int32),
] + [pltpu.VMEM((E2D_C, CW), jnp.float32)] * NBUF \
  + [pltpu.SemaphoreType.DMA] * NBUF


def _sc_segsum_body(src_hbm, gidx_hbm, sidx_hbm, out_hbm,
               acc_sh, gidx_v, sidx_v, r0, r1, s0, s1):
    cid = lax.axis_index("c")
    sid = lax.axis_index("s")
    rows = (r0, r1)
    sems = (s0, s1)
    base = cid * HROWS
    dump = HROWS + sid

    def zrow(r, _):
        for j in range(CW // 16):
            r0[r, pl.ds(j * 16, 16)] = jnp.zeros((16,), jnp.float32)
        return 0

    for c in range(NCH):
        lax.fori_loop(0, SEG, zrow, 0)
        for sb in range(TPR // SEG):
            pltpu.sync_copy(r0.at[pl.ds(0, SEG)],
                            acc_sh.at[pl.ds(sid * TPR + sb * SEG, SEG)])
        plsc.subcore_barrier()

        for seg in range(NSEG):
            sbase = sid * RPT + seg * SEG
            pltpu.sync_copy(gidx_hbm.at[pl.ds(sbase, SEG)], gidx_v)
            pltpu.sync_copy(sidx_hbm.at[pl.ds(sbase, SEG)], sidx_v)

            def clamp_row(j, _):
                for k in range(E2D_C // 16):
                    v = sidx_v[j, pl.ds(k * 16, 16)]
                    rel = v - base
                    ok = jnp.logical_and(rel >= 0, rel < HROWS)
                    sidx_v[j, pl.ds(k * 16, 16)] = jnp.where(ok, rel, dump)
                return 0

            lax.fori_loop(0, SEG, clamp_row, 0)

            for k in range(NBUF):
                pltpu.async_copy(src_hbm.at[c].at[gidx_v.at[k]],
                                 rows[k], sems[k])

            def blk(jb, _):
                for k in range(NBUF):
                    j = jb * NBUF + k
                    pltpu.make_async_copy(
                        src_hbm.at[c].at[gidx_v.at[k]], rows[k], sems[k]).wait()
                    pltpu.sync_copy(rows[k], acc_sh.at[sidx_v.at[j]], add=True)
                    nj = j + NBUF

                    @pl.when(nj < SEG)
                    def _():
                        pltpu.async_copy(src_hbm.at[c].at[gidx_v.at[nj]],
                                         rows[k], sems[k])
                return 0

            lax.fori_loop(0, SEG // NBUF, blk, 0)

        plsc.subcore_barrier()
        pltpu.sync_copy(
            acc_sh.at[pl.ds(sid * TPR, TPR)],
            out_hbm.at[c].at[pl.ds(base + sid * TPR, TPR)])


_sc_segsum = pl.kernel(
    _sc_segsum_body,
    out_type=jax.ShapeDtypeStruct((NCH, N_NODES, CW), jnp.float32),
    mesh=_SC_MESH,
    scratch_types=_SEG_SCRATCH,
)


_TH_BLK = 1200


def _theta0_body(x_ref, pos_ref, tt_ref, ti_ref, out_ref):
    x = x_ref[...] + pos_ref[...]
    out_ref[0] = jnp.dot(x[:, :EMB], tt_ref[...],
                         preferred_element_type=jnp.float32, precision=lax.Precision.HIGHEST)
    out_ref[1] = jnp.dot(x[:, EMB:], ti_ref[...],
                         preferred_element_type=jnp.float32, precision=lax.Precision.HIGHEST)


def _theta0(x, pos_rep, tt, ti):
    return pl.pallas_call(
        _theta0_body,
        grid=(N_NODES // _TH_BLK,),
        in_specs=[
            pl.BlockSpec((_TH_BLK, FW), lambda i: (i, 0)),
            pl.BlockSpec((_TH_BLK, FW), lambda i: (0, 0)),
            pl.BlockSpec((EMB, EMB), lambda i: (0, 0)),
            pl.BlockSpec((EMB, EMB), lambda i: (0, 0)),
        ],
        out_specs=pl.BlockSpec((NCH, _TH_BLK, CW), lambda i: (0, i, 0)),
        out_shape=jax.ShapeDtypeStruct((NCH, N_NODES, CW), jnp.float32),
    )(x, pos_rep, tt, ti)


def _theta1_body(a_ref, dinv_ref, bias_ref, tt_ref, ti_ref, out_ref):
    d = dinv_ref[...]
    xt = a_ref[0] * d + bias_ref[:, :EMB]
    xi = a_ref[1] * d + bias_ref[:, EMB:]
    out_ref[0] = jnp.dot(xt, tt_ref[...], preferred_element_type=jnp.float32, precision=lax.Precision.HIGHEST)
    out_ref[1] = jnp.dot(xi, ti_ref[...], preferred_element_type=jnp.float32, precision=lax.Precision.HIGHEST)


def _theta1(a, dinv_col, bias01, tt, ti):
    return pl.pallas_call(
        _theta1_body,
        grid=(N_NODES // _TH_BLK,),
        in_specs=[
            pl.BlockSpec((NCH, _TH_BLK, CW), lambda i: (0, i, 0)),
            pl.BlockSpec((_TH_BLK, 1), lambda i: (i, 0)),
            pl.BlockSpec((1, FW), lambda i: (0, 0)),
            pl.BlockSpec((EMB, EMB), lambda i: (0, 0)),
            pl.BlockSpec((EMB, EMB), lambda i: (0, 0)),
        ],
        out_specs=pl.BlockSpec((NCH, _TH_BLK, CW), lambda i: (0, i, 0)),
        out_shape=jax.ShapeDtypeStruct((NCH, N_NODES, CW), jnp.float32),
    )(a, dinv_col, bias01, tt, ti)


def _scale_body(e_ref, binv_ref, out_ref):
    b = binv_ref[...]
    for c in range(NCH):
        out_ref[c] = e_ref[c] * b


def _scale_edges(e, binv_col):
    return pl.pallas_call(
        _scale_body,
        grid=(N_NODES // _TH_BLK,),
        in_specs=[
            pl.BlockSpec((NCH, _TH_BLK, CW), lambda i: (0, i, 0)),
            pl.BlockSpec((_TH_BLK, 1), lambda i: (i, 0)),
        ],
        out_specs=pl.BlockSpec((NCH, _TH_BLK, CW), lambda i: (0, i, 0)),
        out_shape=jax.ShapeDtypeStruct((NCH, N_NODES, CW), jnp.float32),
    )(e, binv_col)


def _head_body(xsel_ref, dinv_ref, bias_ref, user_ref,
               w3_ref, b3_ref, wq_ref, bq_ref, wk_ref, bk_ref,
               wv_ref, bv_ref, wo_ref, bo_ref, ln1g_ref, ln1b_ref,
               wf1_ref, bf1_ref, wf2_ref, bf2_ref, ln2g_ref, ln2b_ref,
               dw_ref, db_ref, w4_ref, b4_ref, out_ref):
    x0 = xsel_ref[...] * dinv_ref[...] + bias_ref[...]
    user = user_ref[...]
    tu = jnp.concatenate([x0[:, :EMB], user], axis=1)
    iu = jnp.concatenate([x0[:, EMB:], user], axis=1)
    tiu = jnp.dot(jnp.concatenate([tu, iu], axis=1), w3_ref[...],
                  preferred_element_type=jnp.float32, precision=lax.Precision.HIGHEST) + b3_ref[...]
    q = tiu
    Q = jnp.dot(q, wq_ref[...], preferred_element_type=jnp.float32, precision=lax.Precision.HIGHEST) + bq_ref[...]
    K1 = jnp.dot(tu, wk_ref[...], preferred_element_type=jnp.float32, precision=lax.Precision.HIGHEST) + bk_ref[...]
    K2 = jnp.dot(iu, wk_ref[...], preferred_element_type=jnp.float32, precision=lax.Precision.HIGHEST) + bk_ref[...]
    V1 = jnp.dot(tu, wv_ref[...], preferred_element_type=jnp.float32, precision=lax.Precision.HIGHEST) + bv_ref[...]
    V2 = jnp.dot(iu, wv_ref[...], preferred_element_type=jnp.float32, precision=lax.Precision.HIGHEST) + bv_ref[...]
    r = lax.broadcasted_iota(jnp.int32, (D, HEADS), 0)
    h = lax.broadcasted_iota(jnp.int32, (D, HEADS), 1)
    ind = ((r // DH) == h).astype(jnp.float32)
    rs = 1.0 / jnp.sqrt(jnp.float32(DH))
    s1 = jnp.dot(Q * K1, ind, preferred_element_type=jnp.float32, precision=lax.Precision.HIGHEST) * rs
    s2 = jnp.dot(Q * K2, ind, preferred_element_type=jnp.float32, precision=lax.Precision.HIGHEST) * rs
    m = jnp.maximum(s1, s2)
    e1 = jnp.exp(s1 - m)
    e2 = jnp.exp(s2 - m)
    tot = e1 + e2
    a1 = e1 / tot
    a2 = e2 / tot
    ao = (jnp.dot(a1, ind.T, preferred_element_type=jnp.float32, precision=lax.Precision.HIGHEST) * V1 +
          jnp.dot(a2, ind.T, preferred_element_type=jnp.float32, precision=lax.Precision.HIGHEST) * V2)
    ao = jnp.dot(ao, wo_ref[...], preferred_element_type=jnp.float32, precision=lax.Precision.HIGHEST) + bo_ref[...]

    def ln(x, g, b):
        mu = jnp.mean(x, axis=-1, keepdims=True)
        var = jnp.mean((x - mu) ** 2, axis=-1, keepdims=True)
        return (x - mu) / jnp.sqrt(var + 1e-5) * g + b

    xh = ln(q + ao, ln1g_ref[...], ln1b_ref[...])
    ff = jnp.maximum(
        jnp.dot(xh, wf1_ref[...], preferred_element_type=jnp.float32, precision=lax.Precision.HIGHEST)
        + bf1_ref[...], 0.0)
    ff = jnp.dot(ff, wf2_ref[...], preferred_element_type=jnp.float32, precision=lax.Precision.HIGHEST) + bf2_ref[...]
    xh = ln(xh + ff, ln2g_ref[...], ln2b_ref[...])
    o = jnp.maximum(
        jnp.dot(xh, dw_ref[...], preferred_element_type=jnp.float32, precision=lax.Precision.HIGHEST)
        + db_ref[...], 0.0)
    out_ref[...] = jnp.dot(o, w4_ref[...], preferred_element_type=jnp.float32, precision=lax.Precision.HIGHEST) \
        + b4_ref[...]


def _head(xsel, dinv_sel, bias11, user, p):
    args = (xsel, dinv_sel, bias11, user,
            p['W3'], p['b3'].reshape(1, D),
            p['Wq'], p['bq'].reshape(1, D), p['Wk'], p['bk'].reshape(1, D),
            p['Wv'], p['bv'].reshape(1, D), p['Wo'], p['bo'].reshape(1, D),
            p['ln1_g'].reshape(1, D), p['ln1_b'].reshape(1, D),
            p['Wf1'], p['bf1'].reshape(1, D), p['Wf2'], p['bf2'].reshape(1, D),
            p['ln2_g'].reshape(1, D), p['ln2_b'].reshape(1, D),
            p['dW'], p['db'].reshape(1, D), p['W4'], p['b4'].reshape(1, 1))
    return pl.pallas_call(
        _head_body,
        out_shape=jax.ShapeDtypeStruct((BSZ, 1), jnp.float32),
    )(*args)


def kernel(input, hg_idx, related_items, label, uid, params):
    p = params
    node2d = hg_idx[0].reshape(E2D_R, E2D_C)
    edge2d = hg_idx[1].reshape(E2D_R, E2D_C)
    ri_pad = jnp.concatenate(
        [related_items.astype(jnp.int32),
         jnp.zeros((G2D_R * G2D_C - N_NODES,), jnp.int32)])
    ri2d = ri_pad.reshape(G2D_R, G2D_C)

    P = pl.pallas_call(
        _proj_body,
        grid=(N_ITEMS // _PROJ_BLK,),
        in_specs=[
            pl.BlockSpec((_PROJ_BLK, TEXT_DIM), lambda i: (i, 0)),
            pl.BlockSpec((_PROJ_BLK, IMG_DIM), lambda i: (i, 0)),
            pl.BlockSpec((TEXT_DIM, EMB), lambda i: (0, 0)),
            pl.BlockSpec((1, EMB), lambda i: (0, 0)),
            pl.BlockSpec((IMG_DIM, EMB), lambda i: (0, 0)),
            pl.BlockSpec((1, EMB), lambda i: (0, 0)),
        ],
        out_specs=pl.BlockSpec((_PROJ_BLK, FW), lambda i: (i, 0)),
        out_shape=jax.ShapeDtypeStruct((N_ITEMS, FW), jnp.float32),
    )(p['text_table'], p['img_table'], p['W1'], p['b1'].reshape(1, EMB),
      p['W2'], p['b2'].reshape(1, EMB))

    x = _sc_gather(P, ri2d).reshape(G2D_R * G2D_C, FW)[:N_NODES]
    ones = jnp.ones((E_INC,), jnp.float32)
    dn = jax.ops.segment_sum(ones, hg_idx[0], num_segments=N_NODES)
    be = jax.ops.segment_sum(ones, hg_idx[1], num_segments=N_NODES)
    dinv = jnp.where(dn > 0, 1.0 / dn, 0.0)
    binv = jnp.where(be > 0, 1.0 / be, 0.0)
    dinv_col = dinv.reshape(N_NODES, 1)
    binv_col = binv.reshape(N_NODES, 1)

    y0 = _theta0(x, jnp.asarray(_POS_REP_NP), p['theta_t0'], p['theta_i0'])
    e0 = _sc_segsum(y0, node2d, edge2d)
    e0 = _scale_edges(e0, binv_col)
    a0 = _sc_segsum(e0, edge2d, node2d)

    bias01 = jnp.concatenate([p['bias_t0'], p['bias_i0']]).reshape(1, FW)
    y1 = _theta1(a0, dinv_col, bias01, p['theta_t1'], p['theta_i1'])
    e1 = _sc_segsum(y1, node2d, edge2d)
    e1 = _scale_edges(e1, binv_col)
    a1 = _sc_segsum(e1, edge2d, node2d)

    sel = jnp.arange(BSZ) * LENS
    xsel = jnp.moveaxis(a1[:, ::LENS, :], 0, 1).reshape(BSZ, FW)
    dinv_sel = dinv[sel].reshape(BSZ, 1)
    bias11 = jnp.concatenate([p['bias_t1'], p['bias_i1']]).reshape(1, FW)
    user = p['user_table'][uid]
    return _head(xsel, dinv_sel, bias11, user, p)


# speedup vs baseline: 5.3543x; 1.1014x over previous
import functools

import jax
import jax.numpy as jnp
import numpy as np
from jax import lax
from jax.experimental import pallas as pl
from jax.experimental.pallas import tpu as pltpu
from jax.experimental.pallas import tpu_sc as plsc

BSZ = 64
LENS = 300
EMB = 128
HALF = EMB // 2
D = EMB + HALF
LAYERS = 2
N_NODES = BSZ * LENS
E_INC = 307200
N_ITEMS = 20000
TEXT_DIM = 384
IMG_DIM = 2048
HEADS = 4
DH = D // HEADS

FW = 2 * EMB
NCH = 2
CW = FW // NCH

E2D_R, E2D_C = 3200, 96
RPT = E2D_R // 16
HROWS = N_NODES // 2
ACC_R = HROWS + 16
TPR = HROWS // 16
SEG = 40
NSEG = RPT // SEG
NBUF = 2

G2D_R, G2D_C = 256, 80
GRPW = G2D_R // 32


def _make_pos_rep():
    position = np.arange(LENS)[:, None].astype(np.float64)
    div_term = np.exp(np.arange(0, EMB, 2) * (-np.log(10000.0) / EMB))
    pe = np.zeros((LENS + 1, EMB))
    pe[1:, 0::2] = np.sin(position * div_term)
    pe[1:, 1::2] = np.cos(position * div_term)
    pos = pe[0:LENS].astype(np.float32) * 0.002
    return np.tile(pos, (4, 2))


_POS_REP_NP = _make_pos_rep()

_SC_MESH = plsc.VectorSubcoreMesh(core_axis_name="c", subcore_axis_name="s")


_PROJ_BLK = 400


def _proj_body(text_ref, img_ref, w1_ref, b1_ref, w2_ref, b2_ref, out_ref):
    t = jnp.dot(text_ref[...], w1_ref[...], preferred_element_type=jnp.float32, precision=lax.Precision.HIGHEST)
    i = jnp.dot(img_ref[...], w2_ref[...], preferred_element_type=jnp.float32, precision=lax.Precision.HIGHEST)
    out_ref[:, :EMB] = t + b1_ref[...]
    out_ref[:, EMB:] = i + b2_ref[...]


@functools.partial(
    pl.kernel,
    out_type=jax.ShapeDtypeStruct((G2D_R, G2D_C, FW), jnp.float32),
    mesh=_SC_MESH,
    scratch_types=[
        pltpu.VMEM((GRPW, G2D_C), jnp.int32),
        pltpu.VMEM((G2D_C, FW), jnp.float32),
        pltpu.VMEM((G2D_C, FW), jnp.float32),
        pltpu.SemaphoreType.DMA,
        pltpu.SemaphoreType.DMA,
    ],
)
def _sc_gather(p_hbm, ri_hbm, x_hbm, idx_v, buf0, buf1, sem0, sem1):
    cid = lax.axis_index("c")
    sid = lax.axis_index("s")
    wid = cid * 16 + sid
    pltpu.sync_copy(ri_hbm.at[pl.ds(wid * GRPW, GRPW)], idx_v)
    bufs = (buf0, buf1)
    sems = (sem0, sem1)
    for k in range(2):
        pltpu.async_copy(p_hbm.at[idx_v.at[k]], bufs[k], sems[k])

    def blk(jb, _):
        for k in range(2):
            j = jb * 2 + k
            pltpu.make_async_copy(p_hbm.at[idx_v.at[k]], bufs[k], sems[k]).wait()
            pltpu.sync_copy(bufs[k], x_hbm.at[wid * GRPW + j])
            nj = j + 2

            @pl.when(nj < GRPW)
            def _():
                pltpu.async_copy(p_hbm.at[idx_v.at[nj]], bufs[k], sems[k])
        return 0

    lax.fori_loop(0, GRPW // 2, blk, 0)


_SEG_SCRATCH = [
    pltpu.VMEM_SHARED((ACC_R, CW), jnp.float32),
    pltpu.VMEM((SEG, E2D_C), jnp.int32),
    pltpu.VMEM((SEG, E2D_C), jnp.int32),
] + [pltpu.VMEM((E2D_C, CW), jnp.float32)] * NBUF \
  + [pltpu.SemaphoreType.DMA] * NBUF


def _sc_segsum_body(src_hbm, gidx_hbm, sidx_hbm, out_hbm,
               acc_sh, gidx_v, sidx_v, r0, r1, s0, s1):
    cid = lax.axis_index("c")
    sid = lax.axis_index("s")
    rows = (r0, r1)
    sems = (s0, s1)
    base = cid * HROWS
    dump = HROWS + sid

    def zrow(r, _):
        for j in range(CW // 16):
            r0[r, pl.ds(j * 16, 16)] = jnp.zeros((16,), jnp.float32)
        return 0

    for c in range(NCH):
        lax.fori_loop(0, SEG, zrow, 0)
        for sb in range(TPR // SEG):
            pltpu.sync_copy(r0.at[pl.ds(0, SEG)],
                            acc_sh.at[pl.ds(sid * TPR + sb * SEG, SEG)])
        plsc.subcore_barrier()

        for seg in range(NSEG):
            sbase = sid * RPT + seg * SEG
            pltpu.sync_copy(gidx_hbm.at[pl.ds(sbase, SEG)], gidx_v)
            pltpu.sync_copy(sidx_hbm.at[pl.ds(sbase, SEG)], sidx_v)

            def clamp_row(j, _):
                for k in range(E2D_C // 16):
                    v = sidx_v[j, pl.ds(k * 16, 16)]
                    rel = v - base
                    ok = jnp.logical_and(rel >= 0, rel < HROWS)
                    sidx_v[j, pl.ds(k * 16, 16)] = jnp.where(ok, rel, dump)
                return 0

            lax.fori_loop(0, SEG, clamp_row, 0)

            for k in range(NBUF):
                pltpu.async_copy(src_hbm.at[c].at[gidx_v.at[k]],
                                 rows[k], sems[k])

            def blk(jb, _):
                for k in range(NBUF):
                    j = jb * NBUF + k
                    pltpu.make_async_copy(
                        src_hbm.at[c].at[gidx_v.at[k]], rows[k], sems[k]).wait()
                    pltpu.sync_copy(rows[k], acc_sh.at[sidx_v.at[j]], add=True)
                    nj = j + NBUF

                    @pl.when(nj < SEG)
                    def _():
                        pltpu.async_copy(src_hbm.at[c].at[gidx_v.at[nj]],
                                         rows[k], sems[k])
                return 0

            lax.fori_loop(0, SEG // NBUF, blk, 0)

        plsc.subcore_barrier()
        pltpu.sync_copy(
            acc_sh.at[pl.ds(sid * TPR, TPR)],
            out_hbm.at[c].at[pl.ds(base + sid * TPR, TPR)])


_sc_segsum = pl.kernel(
    _sc_segsum_body,
    out_type=jax.ShapeDtypeStruct((NCH, N_NODES, CW), jnp.float32),
    mesh=_SC_MESH,
    scratch_types=_SEG_SCRATCH,
)


@functools.partial(
    pl.kernel,
    out_type=jax.ShapeDtypeStruct((NCH, BSZ, CW), jnp.float32),
    mesh=_SC_MESH,
    scratch_types=[
        pltpu.VMEM_SHARED((BSZ + 16, CW), jnp.float32),
        pltpu.VMEM((SEG, E2D_C), jnp.int32),
        pltpu.VMEM((SEG, E2D_C), jnp.int32),
    ] + [pltpu.VMEM((E2D_C, CW), jnp.float32)] * NBUF
      + [pltpu.SemaphoreType.DMA] * NBUF,
)
def _sc_seglast(src_hbm, lsidx_hbm, eidx_hbm, out_hbm,
                acc_sh, gidx_v, sidx_v, r0, r1, s0, s1):
    cid = lax.axis_index("c")
    sid = lax.axis_index("s")
    rows = (r0, r1)
    sems = (s0, s1)

    @pl.when(sid == 0)
    def _():
        def zrow(r, _):
            for j in range(CW // 16):
                r0[r, pl.ds(j * 16, 16)] = jnp.zeros((16,), jnp.float32)
            return 0

        lax.fori_loop(0, BSZ + 16, zrow, 0)
        pltpu.sync_copy(r0.at[pl.ds(0, BSZ + 16)], acc_sh)

    plsc.subcore_barrier()

    def do_pass(c):
        for seg in range(NSEG):
            sbase = sid * RPT + seg * SEG
            pltpu.sync_copy(eidx_hbm.at[pl.ds(sbase, SEG)], gidx_v)
            pltpu.sync_copy(lsidx_hbm.at[pl.ds(sbase, SEG)], sidx_v)

            for k in range(NBUF):
                pltpu.async_copy(src_hbm.at[c].at[gidx_v.at[k]],
                                 rows[k], sems[k])

            def blk(jb, _):
                for k in range(NBUF):
                    j = jb * NBUF + k
                    pltpu.make_async_copy(
                        src_hbm.at[c].at[gidx_v.at[k]], rows[k], sems[k]).wait()
                    pltpu.sync_copy(rows[k], acc_sh.at[sidx_v.at[j]], add=True)
                    nj = j + NBUF

                    @pl.when(nj < SEG)
                    def _():
                        pltpu.async_copy(src_hbm.at[c].at[gidx_v.at[nj]],
                                         rows[k], sems[k])
                return 0

            lax.fori_loop(0, SEG // NBUF, blk, 0)

        plsc.subcore_barrier()

        @pl.when(sid == 0)
        def _():
            pltpu.sync_copy(acc_sh.at[pl.ds(0, BSZ)], out_hbm.at[c])

    @pl.when(cid == 0)
    def _():
        do_pass(0)

    @pl.when(cid == 1)
    def _():
        do_pass(1)


_TH_BLK = 1200


def _theta0_body(x_ref, pos_ref, tt_ref, ti_ref, out_ref):
    x = x_ref[...] + pos_ref[...]
    out_ref[0] = jnp.dot(x[:, :EMB], tt_ref[...],
                         preferred_element_type=jnp.float32, precision=lax.Precision.HIGHEST)
    out_ref[1] = jnp.dot(x[:, EMB:], ti_ref[...],
                         preferred_element_type=jnp.float32, precision=lax.Precision.HIGHEST)


def _theta0(x, pos_rep, tt, ti):
    return pl.pallas_call(
        _theta0_body,
        grid=(N_NODES // _TH_BLK,),
        in_specs=[
            pl.BlockSpec((_TH_BLK, FW), lambda i: (i, 0)),
            pl.BlockSpec((_TH_BLK, FW), lambda i: (0, 0)),
            pl.BlockSpec((EMB, EMB), lambda i: (0, 0)),
            pl.BlockSpec((EMB, EMB), lambda i: (0, 0)),
        ],
        out_specs=pl.BlockSpec((NCH, _TH_BLK, CW), lambda i: (0, i, 0)),
        out_shape=jax.ShapeDtypeStruct((NCH, N_NODES, CW), jnp.float32),
    )(x, pos_rep, tt, ti)


def _theta1_body(a_ref, dinv_ref, bias_ref, tt_ref, ti_ref, out_ref):
    d = dinv_ref[...]
    xt = a_ref[0] * d + bias_ref[:, :EMB]
    xi = a_ref[1] * d + bias_ref[:, EMB:]
    out_ref[0] = jnp.dot(xt, tt_ref[...], preferred_element_type=jnp.float32, precision=lax.Precision.HIGHEST)
    out_ref[1] = jnp.dot(xi, ti_ref[...], preferred_element_type=jnp.float32, precision=lax.Precision.HIGHEST)


def _theta1(a, dinv_col, bias01, tt, ti):
    return pl.pallas_call(
        _theta1_body,
        grid=(N_NODES // _TH_BLK,),
        in_specs=[
            pl.BlockSpec((NCH, _TH_BLK, CW), lambda i: (0, i, 0)),
            pl.BlockSpec((_TH_BLK, 1), lambda i: (i, 0)),
            pl.BlockSpec((1, FW), lambda i: (0, 0)),
            pl.BlockSpec((EMB, EMB), lambda i: (0, 0)),
            pl.BlockSpec((EMB, EMB), lambda i: (0, 0)),
        ],
        out_specs=pl.BlockSpec((NCH, _TH_BLK, CW), lambda i: (0, i, 0)),
        out_shape=jax.ShapeDtypeStruct((NCH, N_NODES, CW), jnp.float32),
    )(a, dinv_col, bias01, tt, ti)


def _scale_body(e_ref, binv_ref, out_ref):
    b = binv_ref[...]
    for c in range(NCH):
        out_ref[c] = e_ref[c] * b


def _scale_edges(e, binv_col):
    return pl.pallas_call(
        _scale_body,
        grid=(N_NODES // _TH_BLK,),
        in_specs=[
            pl.BlockSpec((NCH, _TH_BLK, CW), lambda i: (0, i, 0)),
            pl.BlockSpec((_TH_BLK, 1), lambda i: (i, 0)),
        ],
        out_specs=pl.BlockSpec((NCH, _TH_BLK, CW), lambda i: (0, i, 0)),
        out_shape=jax.ShapeDtypeStruct((NCH, N_NODES, CW), jnp.float32),
    )(e, binv_col)


def _head_body(xsel_ref, dinv_ref, bias_ref, user_ref,
               w3_ref, b3_ref, wq_ref, bq_ref, wk_ref, bk_ref,
               wv_ref, bv_ref, wo_ref, bo_ref, ln1g_ref, ln1b_ref,
               wf1_ref, bf1_ref, wf2_ref, bf2_ref, ln2g_ref, ln2b_ref,
               dw_ref, db_ref, w4_ref, b4_ref, out_ref):
    x0 = xsel_ref[...] * dinv_ref[...] + bias_ref[...]
    user = user_ref[...]
    tu = jnp.concatenate([x0[:, :EMB], user], axis=1)
    iu = jnp.concatenate([x0[:, EMB:], user], axis=1)
    tiu = jnp.dot(jnp.concatenate([tu, iu], axis=1), w3_ref[...],
                  preferred_element_type=jnp.float32, precision=lax.Precision.HIGHEST) + b3_ref[...]
    q = tiu
    Q = jnp.dot(q, wq_ref[...], preferred_element_type=jnp.float32, precision=lax.Precision.HIGHEST) + bq_ref[...]
    K1 = jnp.dot(tu, wk_ref[...], preferred_element_type=jnp.float32, precision=lax.Precision.HIGHEST) + bk_ref[...]
    K2 = jnp.dot(iu, wk_ref[...], preferred_element_type=jnp.float32, precision=lax.Precision.HIGHEST) + bk_ref[...]
    V1 = jnp.dot(tu, wv_ref[...], preferred_element_type=jnp.float32, precision=lax.Precision.HIGHEST) + bv_ref[...]
    V2 = jnp.dot(iu, wv_ref[...], preferred_element_type=jnp.float32, precision=lax.Precision.HIGHEST) + bv_ref[...]
    r = lax.broadcasted_iota(jnp.int32, (D, HEADS), 0)
    h = lax.broadcasted_iota(jnp.int32, (D, HEADS), 1)
    ind = ((r // DH) == h).astype(jnp.float32)
    rs = 1.0 / jnp.sqrt(jnp.float32(DH))
    s1 = jnp.dot(Q * K1, ind, preferred_element_type=jnp.float32, precision=lax.Precision.HIGHEST) * rs
    s2 = jnp.dot(Q * K2, ind, preferred_element_type=jnp.float32, precision=lax.Precision.HIGHEST) * rs
    m = jnp.maximum(s1, s2)
    e1 = jnp.exp(s1 - m)
    e2 = jnp.exp(s2 - m)
    tot = e1 + e2
    a1 = e1 / tot
    a2 = e2 / tot
    ao = (jnp.dot(a1, ind.T, preferred_element_type=jnp.float32, precision=lax.Precision.HIGHEST) * V1 +
          jnp.dot(a2, ind.T, preferred_element_type=jnp.float32, precision=lax.Precision.HIGHEST) * V2)
    ao = jnp.dot(ao, wo_ref[...], preferred_element_type=jnp.float32, precision=lax.Precision.HIGHEST) + bo_ref[...]

    def ln(x, g, b):
        mu = jnp.mean(x, axis=-1, keepdims=True)
        var = jnp.mean((x - mu) ** 2, axis=-1, keepdims=True)
        return (x - mu) / jnp.sqrt(var + 1e-5) * g + b

    xh = ln(q + ao, ln1g_ref[...], ln1b_ref[...])
    ff = jnp.maximum(
        jnp.dot(xh, wf1_ref[...], preferred_element_type=jnp.float32, precision=lax.Precision.HIGHEST)
        + bf1_ref[...], 0.0)
    ff = jnp.dot(ff, wf2_ref[...], preferred_element_type=jnp.float32, precision=lax.Precision.HIGHEST) + bf2_ref[...]
    xh = ln(xh + ff, ln2g_ref[...], ln2b_ref[...])
    o = jnp.maximum(
        jnp.dot(xh, dw_ref[...], preferred_element_type=jnp.float32, precision=lax.Precision.HIGHEST)
        + db_ref[...], 0.0)
    out_ref[...] = jnp.dot(o, w4_ref[...], preferred_element_type=jnp.float32, precision=lax.Precision.HIGHEST) \
        + b4_ref[...]


def _head(xsel, dinv_sel, bias11, user, p):
    args = (xsel, dinv_sel, bias11, user,
            p['W3'], p['b3'].reshape(1, D),
            p['Wq'], p['bq'].reshape(1, D), p['Wk'], p['bk'].reshape(1, D),
            p['Wv'], p['bv'].reshape(1, D), p['Wo'], p['bo'].reshape(1, D),
            p['ln1_g'].reshape(1, D), p['ln1_b'].reshape(1, D),
            p['Wf1'], p['bf1'].reshape(1, D), p['Wf2'], p['bf2'].reshape(1, D),
            p['ln2_g'].reshape(1, D), p['ln2_b'].reshape(1, D),
            p['dW'], p['db'].reshape(1, D), p['W4'], p['b4'].reshape(1, 1))
    return pl.pallas_call(
        _head_body,
        out_shape=jax.ShapeDtypeStruct((BSZ, 1), jnp.float32),
    )(*args)


def kernel(input, hg_idx, related_items, label, uid, params):
    p = params
    node2d = hg_idx[0].reshape(E2D_R, E2D_C)
    edge2d = hg_idx[1].reshape(E2D_R, E2D_C)
    ri_pad = jnp.concatenate(
        [related_items.astype(jnp.int32),
         jnp.zeros((G2D_R * G2D_C - N_NODES,), jnp.int32)])
    ri2d = ri_pad.reshape(G2D_R, G2D_C)

    P = pl.pallas_call(
        _proj_body,
        grid=(N_ITEMS // _PROJ_BLK,),
        in_specs=[
            pl.BlockSpec((_PROJ_BLK, TEXT_DIM), lambda i: (i, 0)),
            pl.BlockSpec((_PROJ_BLK, IMG_DIM), lambda i: (i, 0)),
            pl.BlockSpec((TEXT_DIM, EMB), lambda i: (0, 0)),
            pl.BlockSpec((1, EMB), lambda i: (0, 0)),
            pl.BlockSpec((IMG_DIM, EMB), lambda i: (0, 0)),
            pl.BlockSpec((1, EMB), lambda i: (0, 0)),
        ],
        out_specs=pl.BlockSpec((_PROJ_BLK, FW), lambda i: (i, 0)),
        out_shape=jax.ShapeDtypeStruct((N_ITEMS, FW), jnp.float32),
    )(p['text_table'], p['img_table'], p['W1'], p['b1'].reshape(1, EMB),
      p['W2'], p['b2'].reshape(1, EMB))

    x = _sc_gather(P, ri2d).reshape(G2D_R * G2D_C, FW)[:N_NODES]
    ones = jnp.ones((E_INC,), jnp.float32)
    dn = jax.ops.segment_sum(ones, hg_idx[0], num_segments=N_NODES)
    be = jax.ops.segment_sum(ones, hg_idx[1], num_segments=N_NODES)
    dinv = jnp.where(dn > 0, 1.0 / dn, 0.0)
    binv = jnp.where(be > 0, 1.0 / be, 0.0)
    dinv_col = dinv.reshape(N_NODES, 1)
    binv_col = binv.reshape(N_NODES, 1)

    y0 = _theta0(x, jnp.asarray(_POS_REP_NP), p['theta_t0'], p['theta_i0'])
    e0 = _sc_segsum(y0, node2d, edge2d)
    e0 = _scale_edges(e0, binv_col)
    a0 = _sc_segsum(e0, edge2d, node2d)

    bias01 = jnp.concatenate([p['bias_t0'], p['bias_i0']]).reshape(1, FW)
    y1 = _theta1(a0, dinv_col, bias01, p['theta_t1'], p['theta_i1'])
    e1 = _sc_segsum(y1, node2d, edge2d)
    e1 = _scale_edges(e1, binv_col)
    tile_of_row = (jnp.arange(E2D_R, dtype=jnp.int32) // RPT)[:, None]
    lsidx = jnp.where(node2d % LENS == 0, node2d // LENS,
                      BSZ + tile_of_row).astype(jnp.int32)
    a1 = _sc_seglast(e1, lsidx, edge2d)

    sel = jnp.arange(BSZ) * LENS
    xsel = jnp.moveaxis(a1, 0, 1).reshape(BSZ, FW)
    dinv_sel = dinv[sel].reshape(BSZ, 1)
    bias11 = jnp.concatenate([p['bias_t1'], p['bias_i1']]).reshape(1, FW)
    user = p['user_table'][uid]
    return _head(xsel, dinv_sel, bias11, user, p)
